# Initial kernel scaffold; baseline (speedup 1.0000x reference)
import jax
import jax.numpy as jnp
from jax.experimental import pallas as pl


def kernel(x, adj, degree, idx, edge, Wl, bl, W_gamma, W_beta, b_gamma, b_beta, W_add, W_rev):
    raise NotImplementedError("write your pallas kernel here")


# speedup vs baseline: 1.4149x; 1.4149x over previous
import functools

import jax
import jax.numpy as jnp
import numpy as np
from jax.experimental import pallas as pl
from jax.experimental.pallas import tpu as pltpu

DIM_M = 64
D_MAX = 65
OMEGA = 0.1
K_COEF = 1.0
SCALE = DIM_M ** 0.5

N = 10000
F = 128
ROW_BLK = 400
K_BLK = 2000
N_ROW = N // ROW_BLK
N_K = N // K_BLK
IDX_N = 1000
IDX_BLK = 125
N_IDX = IDX_N // IDX_BLK


def _make_pe():
    pos = np.arange(D_MAX)[:, None].astype(np.float64)
    i = np.arange(DIM_M)[None, :].astype(np.float64)
    pe = pos / np.power(10000.0, (i - i % 2) / DIM_M)
    pe[:, 0::2] = np.sin(pe[:, 0::2])
    pe[:, 1::2] = np.cos(pe[:, 1::2])
    return jnp.asarray(pe, jnp.float32)


def _lrelu(v):
    return jnp.where(v >= 0, v, 0.01 * v)


def _prep_kernel(x_ref, wl_ref, bl_ref, h_ref):
    h = jax.lax.dot_general(x_ref[...], wl_ref[...], (((1,), (1,)), ((), ())),
                            preferred_element_type=jnp.float32)
    h_ref[...] = (h + bl_ref[...]) * SCALE


def _main_kernel(adj_ref, h_ref, deg_ref, degfull_ref, pe_ref, wg_ref, wb_ref,
                 bg_ref, bb_ref, wadd_ref, wrev_ref,
                 out_ref, norms_ref):
    r = pl.program_id(0)

    agg = jnp.dot(adj_ref[...], h_ref[...], preferred_element_type=jnp.float32)

    if True:
        deg_i = deg_ref[...]
        deg_f = deg_i.astype(jnp.float32)
        is_zero = deg_f == 0.0
        i_feat = jnp.where(is_zero, 0.0, agg / jnp.where(is_zero, 1.0, deg_f))

        oh = (jax.lax.broadcasted_iota(jnp.int32, (ROW_BLK, D_MAX), 1)
              == deg_i).astype(jnp.float32)
        m_dv = jnp.dot(oh, pe_ref[...], preferred_element_type=jnp.float32)
        gamma = _lrelu(jnp.dot(m_dv, wg_ref[...],
                               preferred_element_type=jnp.float32) + bg_ref[...])
        beta = _lrelu(jnp.dot(m_dv, wb_ref[...],
                              preferred_element_type=jnp.float32) + bb_ref[...])

        g1 = gamma + 1.0
        b_add = g1 * jax.lax.dot_general(
            i_feat, wadd_ref[...], (((1,), (1,)), ((), ())),
            preferred_element_type=jnp.float32) + beta
        b_rev = g1 * jax.lax.dot_general(
            i_feat, wrev_ref[...], (((1,), (1,)), ((), ())),
            preferred_element_type=jnp.float32) + beta

        mean_deg = jnp.sum(degfull_ref[...].astype(jnp.float32)) / np.float32(N)
        big_k = mean_deg * K_COEF
        r_mask = (deg_f < big_k).astype(jnp.float32)

        bias = OMEGA * (r_mask * b_add - (1.0 - r_mask) * b_rev)
        h_row = h_ref[pl.ds(r * ROW_BLK, ROW_BLK), :]
        out_ref[...] = _lrelu((agg + h_row + bias) / (deg_f + 1.0))

        nrm = lambda v: jnp.sqrt(jnp.sum(v * v, axis=1, keepdims=True))
        n_bsel = r_mask * nrm(b_add) + (1.0 - r_mask) * nrm(b_rev)
        norms_ref[...] = jnp.concatenate([n_bsel, nrm(gamma), nrm(beta)], axis=1)


def _loss_kernel(idx_ref, norms_ref, out_ref):
    b = pl.program_id(0)
    idx_blk = idx_ref[b, :]
    oh = (jax.lax.broadcasted_iota(jnp.int32, (IDX_BLK, N), 1)
          == idx_blk[:, None]).astype(jnp.float32)
    part = jnp.dot(oh, norms_ref[...], preferred_element_type=jnp.float32)
    psum = jnp.sum(part, axis=0, keepdims=True)

    @pl.when(b == 0)
    def _():
        out_ref[...] = psum

    @pl.when(b != 0)
    def _():
        out_ref[...] += psum


def kernel(x, adj, degree, idx, edge, Wl, bl, W_gamma, W_beta, b_gamma,
           b_beta, W_add, W_rev):
    pe = _make_pe()
    bl2 = bl.reshape(1, F)

    h = pl.pallas_call(
        _prep_kernel,
        grid=(N_ROW,),
        in_specs=[
            pl.BlockSpec((ROW_BLK, F), lambda r: (r, 0)),
            pl.BlockSpec((F, F), lambda r: (0, 0)),
            pl.BlockSpec((1, F), lambda r: (0, 0)),
        ],
        out_specs=pl.BlockSpec((ROW_BLK, F), lambda r: (r, 0)),
        out_shape=jax.ShapeDtypeStruct((N, F), jnp.float32),
    )(x, Wl, bl2)

    out, norms = pl.pallas_call(
        _main_kernel,
        grid=(N_ROW,),
        in_specs=[
            pl.BlockSpec((ROW_BLK, N), lambda r: (r, 0)),
            pl.BlockSpec((N, F), lambda r: (0, 0)),
            pl.BlockSpec((ROW_BLK, 1), lambda r: (r, 0)),
            pl.BlockSpec((N, 1), lambda r: (0, 0)),
            pl.BlockSpec((D_MAX, DIM_M), lambda r: (0, 0)),
            pl.BlockSpec((DIM_M, F), lambda r: (0, 0)),
            pl.BlockSpec((DIM_M, F), lambda r: (0, 0)),
            pl.BlockSpec((1, F), lambda r: (0, 0)),
            pl.BlockSpec((1, F), lambda r: (0, 0)),
            pl.BlockSpec((F, F), lambda r: (0, 0)),
            pl.BlockSpec((F, F), lambda r: (0, 0)),
        ],
        out_specs=[
            pl.BlockSpec((ROW_BLK, F), lambda r: (r, 0)),
            pl.BlockSpec((ROW_BLK, 3), lambda r: (r, 0)),
        ],
        out_shape=[
            jax.ShapeDtypeStruct((N, F), jnp.float32),
            jax.ShapeDtypeStruct((N, 3), jnp.float32),
        ],
    )(adj, h, degree, degree, pe, W_gamma, W_beta, b_gamma, b_beta,
      W_add, W_rev)

    idx2 = idx.reshape(N_IDX, IDX_BLK)
    sums = pl.pallas_call(
        _loss_kernel,
        grid=(N_IDX,),
        in_specs=[
            pl.BlockSpec((N_IDX, IDX_BLK), lambda b: (0, 0)),
            pl.BlockSpec((N, 3), lambda b: (0, 0)),
        ],
        out_specs=pl.BlockSpec((1, 3), lambda b: (0, 0)),
        out_shape=jax.ShapeDtypeStruct((1, 3), jnp.float32),
    )(idx2, norms)

    inv = np.float32(1.0 / IDX_N)
    l_b = sums[0, 0] * inv
    l_film = (sums[0, 1] + sums[0, 2]) * inv
    return (out, l_b, l_film)
